# baseline (speedup 1.0000x reference)
import functools

import jax
import jax.numpy as jnp
from jax import lax
from jax.experimental import pallas as pl
from jax.experimental.pallas import tpu as pltpu
from jax.experimental.pallas import tpu_sc as plsc

_B = 16384
_D = 16
_MU = 3.5

_INFO = plsc.get_sparse_core_info()
_NC = _INFO.num_cores
_NS = _INFO.num_subcores
_L = _INFO.num_lanes
_NW = _NC * _NS
_BPW = _B // _NW
_NBLK = _BPW // _L


def _svd_bias_body(user_idx, item_idx, eu_w, ei_w, ub_w, ib_w, seg_hbm,
                   out_hbm,
                   idx_u, idx_i, rows_u, rows_i, prod, seg_v, ub_v, ib_v,
                   out_v, acc_sh, s0, s1, s2, s3, s4):
    sid = lax.axis_index("s")
    wid = sid * _NC + lax.axis_index("c")
    base = wid * _BPW

    pltpu.sync_copy(user_idx.at[pl.ds(base, _BPW)], idx_u)
    pltpu.sync_copy(item_idx.at[pl.ds(base, _BPW)], idx_i)

    cu = pltpu.async_copy(eu_w.at[idx_u], rows_u, s0)
    ci = pltpu.async_copy(ei_w.at[idx_i], rows_i, s1)
    cub = pltpu.async_copy(ub_w.at[idx_u], ub_v, s2)
    cib = pltpu.async_copy(ib_w.at[idx_i], ib_v, s3)
    cseg = pltpu.async_copy(seg_hbm, seg_v, s4)
    cub.wait()
    cib.wait()

    def bias_block(blk, carry):
        b0 = blk * _L
        out_v[pl.ds(b0, _L)] = ub_v[pl.ds(b0, _L)] + ib_v[pl.ds(b0, _L)] + _MU
        return carry

    lax.fori_loop(0, _NBLK, bias_block, 0)

    cu.wait()
    ci.wait()

    def prod_block(blk, carry):
        for r in range(_L):
            j = blk * _L + r
            prod[pl.ds(j * _D, _D)] = rows_u[j, :] * rows_i[j, :]
        return carry

    lax.fori_loop(0, _NBLK, prod_block, 0)

    pltpu.sync_copy(out_v, acc_sh.at[sid])
    cseg.wait()
    pltpu.sync_copy(prod, acc_sh.at[sid].at[seg_v], add=True)

    pltpu.sync_copy(acc_sh.at[sid], out_hbm.at[pl.ds(base, _BPW)])


_svd_bias = functools.partial(
    pl.kernel,
    mesh=plsc.VectorSubcoreMesh(core_axis_name="c", subcore_axis_name="s"),
    compiler_params=pltpu.CompilerParams(use_tc_tiling_on_sc=False),
    out_type=jax.ShapeDtypeStruct((_B,), jnp.float32),
    scratch_types=[
        pltpu.VMEM((_BPW,), jnp.int32),
        pltpu.VMEM((_BPW,), jnp.int32),
        pltpu.VMEM((_BPW, _D), jnp.float32),
        pltpu.VMEM((_BPW, _D), jnp.float32),
        pltpu.VMEM((_BPW * _D,), jnp.float32),
        pltpu.VMEM((_BPW * _D,), jnp.int32),
        pltpu.VMEM((_BPW,), jnp.float32),
        pltpu.VMEM((_BPW,), jnp.float32),
        pltpu.VMEM((_BPW,), jnp.float32),
        pltpu.VMEM_SHARED((_NS, _BPW), jnp.float32),
        pltpu.SemaphoreType.DMA,
        pltpu.SemaphoreType.DMA,
        pltpu.SemaphoreType.DMA,
        pltpu.SemaphoreType.DMA,
        pltpu.SemaphoreType.DMA,
    ],
)(_svd_bias_body)


def kernel(user_idx, item_idx, embed_user_w, embed_item_w, user_bias_w, item_bias_w):
    seg = jnp.arange(_BPW * _D, dtype=jnp.int32) // _D
    return _svd_bias(user_idx.astype(jnp.int32), item_idx.astype(jnp.int32),
                     embed_user_w, embed_item_w,
                     user_bias_w.reshape(-1), item_bias_w.reshape(-1), seg)
